# baseline (speedup 1.0000x reference)
import functools

import jax
import jax.numpy as jnp
from jax import lax
from jax.experimental import pallas as pl
from jax.experimental.pallas import tpu as pltpu

_FANOUT = 4
_CHUNK = 2048
_NBUF = 4


def _gather_sum_tile(idx_ref, src, buf, base, m):
    S = m + 1
    for mi in range(m):
        o = base + _FANOUT * mi
        acc = None
        for k in range(_FANOUT):
            ik = pl.multiple_of(idx_ref[o + k], 2)
            slab = src[pl.ds(ik, 2), :]
            acc = slab if acc is None else acc + slab
        buf[mi:mi + 2 * S:S, :] = acc
    return jnp.concatenate([buf[0:m, :], buf[S:S + m, :]], axis=-1)


def _gcn_kernel(idx0_ref, idx1_ref, idx2_ref, feat_hbm,
                w0_ref, b0_ref, w1_ref, wa_ref, wb_ref, b1_ref, b2_ref,
                o_ref, fbuf, h1v, zbuf, tmp0, tmp1, tmp2, tmp3, buf, buf2, buf3, buf4, buf5, buf6, buf7, buf8, sems,
                *, m, nt0, nt1, nt2):
    j = pl.program_id(0)
    n_src = feat_hbm.shape[0]
    chunk = min(_CHUNK, n_src)
    nchunks = n_src // chunk
    bf16 = jnp.bfloat16
    f32 = jnp.float32

    @pl.when(j == 0)
    def _load_interleaved():
        tmps = (tmp0, tmp1, tmp2, tmp3)

        def cp(c):
            return pltpu.make_async_copy(
                feat_hbm.at[pl.ds(c * chunk, chunk), :],
                tmps[c % _NBUF], sems.at[c % _NBUF])

        for c in range(min(_NBUF, nchunks)):
            cp(c).start()
        for c in range(nchunks):
            cp(c).wait()
            if c + _NBUF < nchunks:
                cp(c + _NBUF).start()
            tc = tmps[c % _NBUF]

            def body(r, _, c=c, tc=tc):
                rb = pl.multiple_of(r * 32, 8)
                b = c * (2 * chunk) + r * 64
                for u in range(4):
                    v = tc[pl.ds(rb + 8 * u, 8), :]
                    b0_ = b + 16 * u
                    fbuf[pl.Slice(b0_, 8, 2), :] = v[:, 0:128]
                    fbuf[pl.Slice(b0_ + 1, 8, 2), :] = v[:, 128:256]
                return 0

            lax.fori_loop(0, chunk // 32, body, 0)

    @pl.when(j < nt0)
    def _l0():
        hm = m // 8
        for half, bh in enumerate((buf, buf2, buf3, buf4, buf5, buf6, buf7,
                                   buf8)):
            base = (8 * j + half) * (hm * _FANOUT)
            x = _gather_sum_tile(idx0_ref, fbuf, bh, base, hm)
            y = (jnp.dot(x.astype(bf16), w0_ref[...],
                         preferred_element_type=f32) + b0_ref[...])
            h = jnp.maximum(y, 0.0)
            r0 = 2 * m * j + 2 * hm * half
            h1v[pl.Slice(r0, hm, 2), :] = h[:, :128]
            h1v[pl.Slice(r0 + 1, hm, 2), :] = h[:, 128:]

    @pl.when((j >= nt0) & (j < nt0 + nt1))
    def _l1():
        t = j - nt0
        hm = m // 8
        for half, bh in enumerate((buf, buf2, buf3, buf4, buf5, buf6, buf7,
                                   buf8)):
            base = (8 * t + half) * (hm * _FANOUT)
            x = _gather_sum_tile(idx1_ref, h1v, bh, base, hm)
            y = (jnp.dot(x.astype(bf16), w1_ref[...],
                         preferred_element_type=f32) + b1_ref[...])
            yr = jnp.maximum(y, 0.0)
            z = (jnp.dot(y.astype(bf16), wa_ref[...],
                         preferred_element_type=f32)
                 + jnp.dot(yr.astype(bf16), wb_ref[...],
                           preferred_element_type=f32))
            zbuf[pl.ds(t * m + hm * half, hm), :] = z

    @pl.when(j >= nt0 + nt1)
    def _l2():
        t = j - (nt0 + nt1)
        base = t * (m * _FANOUT)
        bias = b2_ref[...]
        for mi in range(m):
            o = base + _FANOUT * mi
            acc = (zbuf[pl.ds(idx2_ref[o], 1), :]
                   + zbuf[pl.ds(idx2_ref[o + 1], 1), :]
                   + zbuf[pl.ds(idx2_ref[o + 2], 1), :]
                   + zbuf[pl.ds(idx2_ref[o + 3], 1), :])
            o_ref[pl.ds(mi, 1), :] = acc + bias


def _gcn_call(feat, idx0, idx1, idx2, w0, b0, w1, wa, wb, b1, b2, *, m):
    n_src, fin = feat.shape
    n1 = idx0.shape[0] // _FANOUT
    n2 = idx1.shape[0] // _FANOUT
    n3 = idx2.shape[0] // _FANOUT
    fout = wa.shape[1]
    nt0, nt1, nt2 = n1 // m, n2 // m, n3 // m
    chunk = min(_CHUNK, n_src)
    kern = functools.partial(_gcn_kernel, m=m, nt0=nt0, nt1=nt1, nt2=nt2)
    s = nt0 + nt1
    return pl.pallas_call(
        kern,
        out_shape=jax.ShapeDtypeStruct((n3, fout), jnp.float32),
        grid_spec=pltpu.PrefetchScalarGridSpec(
            num_scalar_prefetch=3,
            grid=(nt0 + nt1 + nt2,),
            in_specs=[
                pl.BlockSpec(memory_space=pl.ANY),
                pl.BlockSpec(w0.shape, lambda j, i0, i1, i2: (0, 0)),
                pl.BlockSpec(b0.shape, lambda j, i0, i1, i2: (0, 0)),
                pl.BlockSpec(w1.shape, lambda j, i0, i1, i2: (0, 0)),
                pl.BlockSpec(wa.shape, lambda j, i0, i1, i2: (0, 0)),
                pl.BlockSpec(wb.shape, lambda j, i0, i1, i2: (0, 0)),
                pl.BlockSpec(b1.shape, lambda j, i0, i1, i2: (0, 0)),
                pl.BlockSpec(b2.shape, lambda j, i0, i1, i2: (0, 0)),
            ],
            out_specs=pl.BlockSpec(
                (m, fout),
                lambda j, i0, i1, i2: (jnp.maximum(j - s, 0), 0)),
            scratch_shapes=[
                pltpu.VMEM((2 * n_src, 128), jnp.float32),
                pltpu.VMEM((2 * n1, 128), jnp.float32),
                pltpu.VMEM((n2, fout), jnp.float32),
                pltpu.VMEM((chunk, fin), jnp.float32),
                pltpu.VMEM((chunk, fin), jnp.float32),
                pltpu.VMEM((chunk, fin), jnp.float32),
                pltpu.VMEM((chunk, fin), jnp.float32),
                pltpu.VMEM((m // 4 + 2, 128), jnp.float32),
                pltpu.VMEM((m // 4 + 2, 128), jnp.float32),
                pltpu.VMEM((m // 4 + 2, 128), jnp.float32),
                pltpu.VMEM((m // 4 + 2, 128), jnp.float32),
                pltpu.VMEM((m // 4 + 2, 128), jnp.float32),
                pltpu.VMEM((m // 4 + 2, 128), jnp.float32),
                pltpu.VMEM((m // 4 + 2, 128), jnp.float32),
                pltpu.VMEM((m // 4 + 2, 128), jnp.float32),
                pltpu.SemaphoreType.DMA((4,)),
            ],
        ),
        compiler_params=pltpu.CompilerParams(
            dimension_semantics=("arbitrary",),
            vmem_limit_bytes=58 << 20,
        ),
    )(idx0, idx1, idx2, feat, w0, b0, w1, wa, wb, b1, b2)


def kernel(features, w0, b0, w1, b1, w2, b2, nbr0, nbr1, nbr2):
    f32 = jnp.float32
    bf16 = jnp.bfloat16
    fmid = w1.shape[0]

    idx0 = (nbr0.astype(jnp.int32) * 2).reshape(-1)
    idx1 = (nbr1.astype(jnp.int32) * 2).reshape(-1)
    idx2 = nbr2.astype(jnp.int32).reshape(-1)
    w0s = (w0.astype(f32) / _FANOUT).astype(bf16)
    w1s = (w1.astype(f32) / _FANOUT).astype(bf16)
    wa = (w2[:fmid].astype(f32) / _FANOUT).astype(bf16)
    wb = (w2[fmid:].astype(f32) / _FANOUT).astype(bf16)
    out = _gcn_call(features.astype(f32), idx0, idx1, idx2,
                    w0s, b0.astype(f32).reshape(1, -1),
                    w1s, wa, wb,
                    b1.astype(f32).reshape(1, -1),
                    b2.astype(f32).reshape(1, -1), m=2048)
    return out.astype(f32)


# speedup vs baseline: 1.0095x; 1.0095x over previous
import functools

import jax
import jax.numpy as jnp
from jax import lax
from jax.experimental import pallas as pl
from jax.experimental.pallas import tpu as pltpu

_FANOUT = 4
_CHUNK = 2048
_NBUF = 4


def _gather_sum_tile(idx_ref, src, buf, base, m):
    S = m + 1
    for mi in range(m):
        o = base + _FANOUT * mi
        acc = None
        for k in range(_FANOUT):
            ik = pl.multiple_of(idx_ref[o + k], 2)
            slab = src[pl.ds(ik, 2), :]
            acc = slab if acc is None else acc + slab
        buf[mi:mi + 2 * S:S, :] = acc
    return jnp.concatenate([buf[0:m, :], buf[S:S + m, :]], axis=-1)


def _gcn_kernel(idx0_ref, idx1_ref, idx2_ref, feat_hbm,
                w0_ref, b0_ref, w1_ref, wa_ref, wb_ref, b1_ref, b2_ref,
                o_ref, fbuf, h1v, zbuf, tmp0, tmp1, tmp2, tmp3, buf, buf2, buf3, buf4, sems,
                *, m, nt0, nt1, nt2):
    j = pl.program_id(0)
    n_src = feat_hbm.shape[0]
    chunk = min(_CHUNK, n_src)
    nchunks = n_src // chunk
    bf16 = jnp.bfloat16
    f32 = jnp.float32

    @pl.when(j == 0)
    def _load_interleaved():
        tmps = (tmp0, tmp1, tmp2, tmp3)

        def cp(c):
            return pltpu.make_async_copy(
                feat_hbm.at[pl.ds(c * chunk, chunk), :],
                tmps[c % _NBUF], sems.at[c % _NBUF])

        for c in range(min(_NBUF, nchunks)):
            cp(c).start()
        for c in range(nchunks):
            cp(c).wait()
            if c + _NBUF < nchunks:
                cp(c + _NBUF).start()
            tc = tmps[c % _NBUF]

            def body(r, _, c=c, tc=tc):
                rb = pl.multiple_of(r * 32, 8)
                b = c * (2 * chunk) + r * 64
                for u in range(4):
                    v = tc[pl.ds(rb + 8 * u, 8), :]
                    b0_ = b + 16 * u
                    fbuf[pl.Slice(b0_, 8, 2), :] = v[:, 0:128]
                    fbuf[pl.Slice(b0_ + 1, 8, 2), :] = v[:, 128:256]
                return 0

            lax.fori_loop(0, chunk // 32, body, 0)

    @pl.when(j < nt0)
    def _l0():
        hm = m // 4
        for half, bh in ((0, buf), (1, buf2), (2, buf3), (3, buf4)):
            base = (4 * j + half) * (hm * _FANOUT)
            x = _gather_sum_tile(idx0_ref, fbuf, bh, base, hm)
            y = (jnp.dot(x.astype(bf16), w0_ref[...],
                         preferred_element_type=f32) + b0_ref[...])
            h = jnp.maximum(y, 0.0)
            r0 = 2 * m * j + 2 * hm * half
            h1v[pl.Slice(r0, hm, 2), :] = h[:, :128]
            h1v[pl.Slice(r0 + 1, hm, 2), :] = h[:, 128:]

    @pl.when((j >= nt0) & (j < nt0 + nt1))
    def _l1():
        t = j - nt0
        hm = m // 4
        for half, bh in ((0, buf), (1, buf2), (2, buf3), (3, buf4)):
            base = (4 * t + half) * (hm * _FANOUT)
            x = _gather_sum_tile(idx1_ref, h1v, bh, base, hm)
            y = (jnp.dot(x.astype(bf16), w1_ref[...],
                         preferred_element_type=f32) + b1_ref[...])
            yr = jnp.maximum(y, 0.0)
            z = (jnp.dot(y.astype(bf16), wa_ref[...],
                         preferred_element_type=f32)
                 + jnp.dot(yr.astype(bf16), wb_ref[...],
                           preferred_element_type=f32))
            zbuf[pl.ds(t * m + hm * half, hm), :] = z

    @pl.when(j >= nt0 + nt1)
    def _l2():
        t = j - (nt0 + nt1)
        base = t * (m * _FANOUT)
        bias = b2_ref[...]
        for mi in range(m):
            o = base + _FANOUT * mi
            acc = (zbuf[pl.ds(idx2_ref[o], 1), :]
                   + zbuf[pl.ds(idx2_ref[o + 1], 1), :]
                   + zbuf[pl.ds(idx2_ref[o + 2], 1), :]
                   + zbuf[pl.ds(idx2_ref[o + 3], 1), :])
            o_ref[pl.ds(mi, 1), :] = acc + bias


def _gcn_call(feat, idx0, idx1, idx2, w0, b0, w1, wa, wb, b1, b2, *, m):
    n_src, fin = feat.shape
    n1 = idx0.shape[0] // _FANOUT
    n2 = idx1.shape[0] // _FANOUT
    n3 = idx2.shape[0] // _FANOUT
    fout = wa.shape[1]
    nt0, nt1, nt2 = n1 // m, n2 // m, n3 // m
    chunk = min(_CHUNK, n_src)
    kern = functools.partial(_gcn_kernel, m=m, nt0=nt0, nt1=nt1, nt2=nt2)
    s = nt0 + nt1
    return pl.pallas_call(
        kern,
        out_shape=jax.ShapeDtypeStruct((n3, fout), jnp.float32),
        grid_spec=pltpu.PrefetchScalarGridSpec(
            num_scalar_prefetch=3,
            grid=(nt0 + nt1 + nt2,),
            in_specs=[
                pl.BlockSpec(memory_space=pl.ANY),
                pl.BlockSpec(w0.shape, lambda j, i0, i1, i2: (0, 0)),
                pl.BlockSpec(b0.shape, lambda j, i0, i1, i2: (0, 0)),
                pl.BlockSpec(w1.shape, lambda j, i0, i1, i2: (0, 0)),
                pl.BlockSpec(wa.shape, lambda j, i0, i1, i2: (0, 0)),
                pl.BlockSpec(wb.shape, lambda j, i0, i1, i2: (0, 0)),
                pl.BlockSpec(b1.shape, lambda j, i0, i1, i2: (0, 0)),
                pl.BlockSpec(b2.shape, lambda j, i0, i1, i2: (0, 0)),
            ],
            out_specs=pl.BlockSpec(
                (m, fout),
                lambda j, i0, i1, i2: (jnp.maximum(j - s, 0), 0)),
            scratch_shapes=[
                pltpu.VMEM((2 * n_src, 128), jnp.float32),
                pltpu.VMEM((2 * n1, 128), jnp.float32),
                pltpu.VMEM((n2, fout), jnp.float32),
                pltpu.VMEM((chunk, fin), jnp.float32),
                pltpu.VMEM((chunk, fin), jnp.float32),
                pltpu.VMEM((chunk, fin), jnp.float32),
                pltpu.VMEM((chunk, fin), jnp.float32),
                pltpu.VMEM((m // 2 + 2, 128), jnp.float32),
                pltpu.VMEM((m // 2 + 2, 128), jnp.float32),
                pltpu.VMEM((m // 2 + 2, 128), jnp.float32),
                pltpu.VMEM((m // 2 + 2, 128), jnp.float32),
                pltpu.SemaphoreType.DMA((4,)),
            ],
        ),
        compiler_params=pltpu.CompilerParams(
            dimension_semantics=("arbitrary",),
            vmem_limit_bytes=58 << 20,
        ),
    )(idx0, idx1, idx2, feat, w0, b0, w1, wa, wb, b1, b2)


def kernel(features, w0, b0, w1, b1, w2, b2, nbr0, nbr1, nbr2):
    f32 = jnp.float32
    bf16 = jnp.bfloat16
    fmid = w1.shape[0]

    idx0 = (nbr0.astype(jnp.int32) * 2).reshape(-1)
    idx1 = (nbr1.astype(jnp.int32) * 2).reshape(-1)
    idx2 = nbr2.astype(jnp.int32).reshape(-1)
    w0s = (w0.astype(f32) / _FANOUT).astype(bf16)
    w1s = (w1.astype(f32) / _FANOUT).astype(bf16)
    wa = (w2[:fmid].astype(f32) / _FANOUT).astype(bf16)
    wb = (w2[fmid:].astype(f32) / _FANOUT).astype(bf16)
    out = _gcn_call(features.astype(f32), idx0, idx1, idx2,
                    w0s, b0.astype(f32).reshape(1, -1),
                    w1s, wa, wb,
                    b1.astype(f32).reshape(1, -1),
                    b2.astype(f32).reshape(1, -1), m=2048)
    return out.astype(f32)
